# Initial kernel scaffold; baseline (speedup 1.0000x reference)
import jax
import jax.numpy as jnp
from jax.experimental import pallas as pl


def kernel(table, idx):
    raise NotImplementedError("write your pallas kernel here")


# speedup vs baseline: 9.7519x; 9.7519x over previous
import jax
import jax.numpy as jnp
from jax.experimental import pallas as pl

_K = 32
_ROW_BLOCK = 200
_HI_BITS = 0x3F800800
_SEARCH_ITERS = 31


def _graph_block_kernel(wb_ref, wt_ref, out_ref):
    wb = wb_ref[:]
    wt = wt_ref[:]
    s = jax.lax.dot_general(wb, wt, (((1,), (0,)), ((), ())),
                            preferred_element_type=jnp.float32)
    nb = jnp.sqrt(jnp.sum(wb * wb, axis=1, keepdims=True))
    nall = jnp.sqrt(jnp.sum(wt * wt, axis=0, keepdims=True))
    c = s / (nb * nall)
    a = jnp.abs(c)
    bits = jax.lax.bitcast_convert_type(a, jnp.int32)

    rb = a.shape[0]
    lo0 = jnp.zeros((rb, 1), jnp.int32)
    hi0 = jnp.full((rb, 1), _HI_BITS, jnp.int32)

    def body(_, carry):
        lo, hi = carry
        mid = lo + (hi - lo + 1) // 2
        cnt = jnp.sum((bits >= mid).astype(jnp.int32), axis=1, keepdims=True)
        ge = cnt >= _K
        return jnp.where(ge, mid, lo), jnp.where(ge, hi, mid - 1)

    lo, _ = jax.lax.fori_loop(0, _SEARCH_ITERS, body, (lo0, hi0))
    thr = jax.lax.bitcast_convert_type(lo, jnp.float32)
    out_ref[:] = jnp.where(a >= thr, c, 0.0)


def kernel(table, idx):
    weights = jnp.take(table, idx, axis=0)
    n, d = weights.shape
    wt = weights.T
    rb = _ROW_BLOCK
    return pl.pallas_call(
        _graph_block_kernel,
        grid=(n // rb,),
        in_specs=[
            pl.BlockSpec((rb, d), lambda i: (i, 0)),
            pl.BlockSpec((d, n), lambda i: (0, 0)),
        ],
        out_specs=pl.BlockSpec((rb, n), lambda i: (i, 0)),
        out_shape=jax.ShapeDtypeStruct((n, n), jnp.float32),
    )(weights, wt)


# speedup vs baseline: 12.4775x; 1.2795x over previous
import jax
import jax.numpy as jnp
from jax.experimental import pallas as pl

_K = 32
_ROW_BLOCK = 200
_HI_BITS = 0x3F800800
_MAX_ITERS = 48
_HALF_NORMAL = 1.2533141
_Z_TAIL = 2.948


def _graph_block_kernel(wb_ref, wt_ref, out_ref):
    wb = wb_ref[:]
    wt = wt_ref[:]
    s = jax.lax.dot_general(wb, wt, (((1,), (0,)), ((), ())),
                            preferred_element_type=jnp.float32)
    nb = jnp.sqrt(jnp.sum(wb * wb, axis=1, keepdims=True))
    nall = jnp.sqrt(jnp.sum(wt * wt, axis=0, keepdims=True))
    c = s / (nb * nall)
    a = jnp.abs(c)
    bits = jax.lax.bitcast_convert_type(a, jnp.int32)

    rb = a.shape[0]
    n = a.shape[1]
    kf = jnp.float32(_K)
    sigma = (jnp.sum(a, axis=1, keepdims=True) / n) * _HALF_NORMAL

    def count_ge(t_bits):
        return jnp.sum((bits >= t_bits).astype(jnp.float32), axis=1,
                       keepdims=True)

    lo0 = jnp.zeros((rb, 1), jnp.int32)
    hi0 = jnp.full((rb, 1), _HI_BITS, jnp.int32)
    t_prev0 = sigma * _Z_TAIL
    f_prev0 = jnp.full((rb, 1), kf, jnp.float32)
    found0 = jnp.zeros((rb, 1), jnp.int32)
    thr0 = jnp.zeros((rb, 1), jnp.int32)
    i0 = jnp.int32(0)

    def cond(carry):
        i, lo, hi, _, _, found, _ = carry
        done = jnp.all((found > 0) | (lo >= hi))
        return (i < _MAX_ITERS) & jnp.logical_not(done)

    def body(carry):
        i, lo, hi, t_prev, f_prev, found, thr = carry
        t_model = jnp.sqrt(t_prev * t_prev +
                           2.0 * sigma * sigma * jnp.log(f_prev / kf))
        tm_bits = jax.lax.bitcast_convert_type(t_model, jnp.int32)
        bisect = lo + (hi - lo + 1) // 2
        use_model = ((jax.lax.rem(i, jnp.int32(2)) == 0)
                     & (tm_bits > lo) & (tm_bits <= hi))
        t_bits = jnp.where(use_model, tm_bits, bisect)
        cnt = count_ge(t_bits)
        ge = cnt >= kf
        lo = jnp.where(ge, t_bits, lo)
        hi = jnp.where(ge, hi, t_bits - 1)
        hit = (cnt == kf) & (found == 0)
        thr = jnp.where(hit, t_bits, thr)
        found = jnp.where(hit, 1, found)
        t_f = jax.lax.bitcast_convert_type(t_bits, jnp.float32)
        return (i + 1, lo, hi, t_f, jnp.maximum(cnt, 0.5), found, thr)

    _, lo, _, _, _, found, thr = jax.lax.while_loop(
        cond, body, (i0, lo0, hi0, t_prev0, f_prev0, found0, thr0))
    thr = jnp.where(found > 0, thr, lo)
    out_ref[:] = jnp.where(bits >= thr, c, 0.0)


def kernel(table, idx):
    weights = jnp.take(table, idx, axis=0)
    n, d = weights.shape
    wt = weights.T
    rb = _ROW_BLOCK
    return pl.pallas_call(
        _graph_block_kernel,
        grid=(n // rb,),
        in_specs=[
            pl.BlockSpec((rb, d), lambda i: (i, 0)),
            pl.BlockSpec((d, n), lambda i: (0, 0)),
        ],
        out_specs=pl.BlockSpec((rb, n), lambda i: (i, 0)),
        out_shape=jax.ShapeDtypeStruct((n, n), jnp.float32),
    )(weights, wt)


# speedup vs baseline: 12.4815x; 1.0003x over previous
import jax
import jax.numpy as jnp
from jax.experimental import pallas as pl
from jax.experimental.pallas import tpu as pltpu

_K = 32
_ROW_BLOCK = 200
_HI_BITS = 0x3F800800
_MAX_ITERS = 48
_HALF_NORMAL = 1.2533141
_Z_TAIL = 2.948


def _graph_block_kernel(wb_ref, wt_ref, out_ref):
    wb = wb_ref[:]
    wt = wt_ref[:]
    s = jax.lax.dot_general(wb, wt, (((1,), (0,)), ((), ())),
                            preferred_element_type=jnp.float32)
    nb = jnp.sqrt(jnp.sum(wb * wb, axis=1, keepdims=True))
    nall = jnp.sqrt(jnp.sum(wt * wt, axis=0, keepdims=True))
    c = s / (nb * nall)
    a = jnp.abs(c)
    bits = jax.lax.bitcast_convert_type(a, jnp.int32)

    rb = a.shape[0]
    n = a.shape[1]
    kf = jnp.float32(_K)
    sigma = (jnp.sum(a, axis=1, keepdims=True) / n) * _HALF_NORMAL

    def count_ge(t_bits):
        return jnp.sum((bits >= t_bits).astype(jnp.float32), axis=1,
                       keepdims=True)

    lo0 = jnp.zeros((rb, 1), jnp.int32)
    hi0 = jnp.full((rb, 1), _HI_BITS, jnp.int32)
    t_prev0 = sigma * _Z_TAIL
    f_prev0 = jnp.full((rb, 1), kf, jnp.float32)
    found0 = jnp.zeros((rb, 1), jnp.int32)
    thr0 = jnp.zeros((rb, 1), jnp.int32)
    i0 = jnp.int32(0)

    def cond(carry):
        i, lo, hi, _, _, found, _ = carry
        done = jnp.all((found > 0) | (lo >= hi))
        return (i < _MAX_ITERS) & jnp.logical_not(done)

    def body(carry):
        i, lo, hi, t_prev, f_prev, found, thr = carry
        t_model = jnp.sqrt(t_prev * t_prev +
                           2.0 * sigma * sigma * jnp.log(f_prev / kf))
        tm_bits = jax.lax.bitcast_convert_type(t_model, jnp.int32)
        bisect = lo + (hi - lo + 1) // 2
        use_model = ((jax.lax.rem(i, jnp.int32(2)) == 0)
                     & (tm_bits > lo) & (tm_bits <= hi))
        t_bits = jnp.where(use_model, tm_bits, bisect)
        cnt = count_ge(t_bits)
        ge = cnt >= kf
        lo = jnp.where(ge, t_bits, lo)
        hi = jnp.where(ge, hi, t_bits - 1)
        hit = (cnt == kf) & (found == 0)
        thr = jnp.where(hit, t_bits, thr)
        found = jnp.where(hit, 1, found)
        t_f = jax.lax.bitcast_convert_type(t_bits, jnp.float32)
        return (i + 1, lo, hi, t_f, jnp.maximum(cnt, 0.5), found, thr)

    _, lo, _, _, _, found, thr = jax.lax.while_loop(
        cond, body, (i0, lo0, hi0, t_prev0, f_prev0, found0, thr0))
    thr = jnp.where(found > 0, thr, lo)
    out_ref[:] = jnp.where(bits >= thr, c, 0.0)


def kernel(table, idx):
    weights = jnp.take(table, idx, axis=0)
    n, d = weights.shape
    wt = weights.T
    rb = _ROW_BLOCK
    return pl.pallas_call(
        _graph_block_kernel,
        grid=(n // rb,),
        in_specs=[
            pl.BlockSpec((rb, d), lambda i: (i, 0)),
            pl.BlockSpec((d, n), lambda i: (0, 0)),
        ],
        out_specs=pl.BlockSpec((rb, n), lambda i: (i, 0)),
        out_shape=jax.ShapeDtypeStruct((n, n), jnp.float32),
        compiler_params=pltpu.CompilerParams(
            dimension_semantics=("parallel",)),
    )(weights, wt)
